# baseline (speedup 1.0000x reference)
import jax
import jax.numpy as jnp
from jax import lax
from jax.experimental import pallas as pl
from jax.experimental.pallas import tpu as pltpu
from jax.experimental.pallas import tpu_sc as plsc

_BLOCK_ROWS = 16384

_NUM_CORES = 2
_NUM_SUBCORES = 16
_NW = _NUM_CORES * _NUM_SUBCORES
_CHUNK = 504


def _tc_copy_block(x_ref, o_ref):
    o_ref[...] = x_ref[...]


def _sc_copy_body(half_rows, rows_per_w):
    nfull = rows_per_w // _CHUNK
    tail = rows_per_w - nfull * _CHUNK
    sizes = [_CHUNK] * nfull + ([tail] if tail else [])
    starts = [i * _CHUNK for i in range(len(sizes))]
    nchunks = len(sizes)

    def body(x_hbm, o_hbm, buf0, buf1, isem0, isem1, osem0, osem1):
        bufs = (buf0, buf1)
        isems = (isem0, isem1)
        osems = (osem0, osem1)
        c = lax.axis_index("c")
        s = lax.axis_index("s")
        base = (s * _NUM_CORES + c) * rows_per_w

        def in_copy(i):
            b = i % 2
            return pltpu.make_async_copy(
                x_hbm.at[pl.ds(half_rows + base + starts[i], sizes[i])],
                bufs[b].at[pl.ds(0, sizes[i])], isems[b])

        def out_copy(i):
            b = i % 2
            return pltpu.make_async_copy(
                bufs[b].at[pl.ds(0, sizes[i])],
                o_hbm.at[pl.ds(base + starts[i], sizes[i])], osems[b])

        in_copy(0).start()
        for i in range(nchunks):
            if i + 1 < nchunks:
                if i >= 1:
                    out_copy(i - 1).wait()
                in_copy(i + 1).start()
            in_copy(i).wait()
            out_copy(i).start()
        if nchunks >= 2:
            out_copy(nchunks - 2).wait()
        out_copy(nchunks - 1).wait()

    return body


def kernel(data, partitions):
    del partitions
    n, d = data.shape
    half = n // 2

    top = pl.pallas_call(
        _tc_copy_block,
        grid=(half // _BLOCK_ROWS,),
        in_specs=[pl.BlockSpec((_BLOCK_ROWS, d), lambda i: (i, 0))],
        out_specs=pl.BlockSpec((_BLOCK_ROWS, d), lambda i: (i, 0)),
        out_shape=jax.ShapeDtypeStruct((half, d), data.dtype),
    )(data)

    mesh = plsc.VectorSubcoreMesh(
        core_axis_name="c", subcore_axis_name="s",
        num_cores=_NUM_CORES, num_subcores=_NUM_SUBCORES)
    sc_copy = pl.kernel(
        _sc_copy_body(half, half // _NW),
        out_type=jax.ShapeDtypeStruct((half, d), data.dtype),
        mesh=mesh,
        scratch_types=(
            [pltpu.VMEM((_CHUNK, d), jnp.float32)] * 2
            + [pltpu.SemaphoreType.DMA] * 4),
    )
    bottom = sc_copy(data)

    return jnp.concatenate([top, bottom], axis=0)


# speedup vs baseline: 1.2277x; 1.2277x over previous
import jax
import jax.numpy as jnp
from jax import lax
from jax.experimental import pallas as pl
from jax.experimental.pallas import tpu as pltpu

_CHUNK_ROWS = 8192
_NSLOTS = 8


def _make_copy_kernel(nchunks):
    ngroups = nchunks // _NSLOTS

    def _copy(x_hbm, o_hbm, buf, *sems):
        in_sems, out_sems = sems[:_NSLOTS], sems[_NSLOTS:]

        def in_start(i, b):
            pltpu.async_copy(
                x_hbm.at[pl.ds(i * _CHUNK_ROWS, _CHUNK_ROWS)],
                buf.at[b], in_sems[b], priority=b % 2)

        def in_wait(i, b):
            pltpu.make_async_copy(
                x_hbm.at[pl.ds(i * _CHUNK_ROWS, _CHUNK_ROWS)],
                buf.at[b], in_sems[b]).wait()

        def out_start(i, b):
            pltpu.async_copy(
                buf.at[b],
                o_hbm.at[pl.ds(i * _CHUNK_ROWS, _CHUNK_ROWS)], out_sems[b],
                priority=b % 2)

        def out_wait(i, b):
            pltpu.make_async_copy(
                buf.at[b],
                o_hbm.at[pl.ds(i * _CHUNK_ROWS, _CHUNK_ROWS)],
                out_sems[b]).wait()

        for b in range(_NSLOTS):
            in_start(b, b)

        def body(g, carry):
            i0 = g * _NSLOTS
            for b in range(_NSLOTS):
                in_wait(i0 + b, b)
                out_start(i0 + b, b)
            for b in range(_NSLOTS):
                out_wait(i0 + b, b)

                @pl.when(i0 + b + _NSLOTS < nchunks)
                def _():
                    in_start(i0 + b + _NSLOTS, b)

            return carry

        lax.fori_loop(0, ngroups, body, 0)

    return _copy


def kernel(data, partitions):
    del partitions
    n, d = data.shape
    nchunks = n // _CHUNK_ROWS
    return pl.pallas_call(
        _make_copy_kernel(nchunks),
        in_specs=[pl.BlockSpec(memory_space=pl.ANY)],
        out_specs=pl.BlockSpec(memory_space=pl.ANY),
        out_shape=jax.ShapeDtypeStruct((n, d), data.dtype),
        scratch_shapes=(
            [pltpu.VMEM((_NSLOTS, _CHUNK_ROWS, d), jnp.float32)]
            + [pltpu.SemaphoreType.DMA] * (2 * _NSLOTS)),
    )(data)
